# Initial kernel scaffold; baseline (speedup 1.0000x reference)
import jax
import jax.numpy as jnp
from jax.experimental import pallas as pl


def kernel(x, edge_index, edge_attr, W_self1, b_self1, W_neigh1, b_neigh1, W_self2, b_self2, W_neigh2, b_neigh2):
    raise NotImplementedError("write your pallas kernel here")


# speedup vs baseline: 6.8448x; 6.8448x over previous
import jax
import jax.numpy as jnp
from jax import lax
from jax.experimental import pallas as pl
from jax.experimental.pallas import tpu as pltpu
from jax.experimental.pallas import tpu_sc as plsc

N_NODES = 10000
N_EDGES = 320000
D = 128
DE = 16
HF = 64

NC = 2
NS = 16
EPT = N_EDGES // NS
CH = 125
NCH = EPT // CH
NPT = N_NODES // NS
CW = 8


def _make_sc_pass(with_meta: bool):
    out_type = [jax.ShapeDtypeStruct((NC, N_NODES, HF), jnp.float32)]
    if with_meta:
        out_type += [
            jax.ShapeDtypeStruct((N_NODES, DE), jnp.float32),
            jax.ShapeDtypeStruct((N_NODES, CW), jnp.float32),
        ]
    scratch = [
        pltpu.VMEM((NCH, CH), jnp.int32),
        pltpu.VMEM((NCH, CH), jnp.int32),
        pltpu.VMEM((CH, HF), jnp.float32),
        pltpu.VMEM_SHARED((N_NODES, HF), jnp.float32),
        pltpu.SemaphoreType.DMA,
    ]
    if with_meta:
        scratch += [
            pltpu.VMEM((CH, DE), jnp.float32),
            pltpu.VMEM((128, CW), jnp.float32),
            pltpu.VMEM_SHARED((N_NODES, DE), jnp.float32),
            pltpu.VMEM_SHARED((N_NODES, CW), jnp.float32),
        ]

    def body(x2_hbm, src_hbm, dst_hbm, *rest):
        if with_meta:
            (ea_hbm, onesz_hbm, out_h, out_e, out_c,
             idx_s, idx_d, rows, acc_h, sem,
             eabuf, ones, acc_e, acc_c) = rest
        else:
            (out_h, idx_s, idx_d, rows, acc_h, sem) = rest

        cid = lax.axis_index("c")
        sid = lax.axis_index("s")

        pltpu.sync_copy(src_hbm.at[cid, sid], idx_s)
        pltpu.sync_copy(dst_hbm.at[sid], idx_d)

        z = jnp.zeros((16,), jnp.float32)

        def zrow(i, c):
            for k in range(HF // 16):
                rows[i, pl.ds(k * 16, 16)] = z
            return c

        lax.fori_loop(0, CH, zrow, 0)
        for k in range(NPT // CH):
            pltpu.sync_copy(rows, acc_h.at[pl.ds(sid * NPT + k * CH, CH)])

        if with_meta:
            @pl.when(cid == 0)
            def _():
                def zea(i, c):
                    eabuf[i, pl.ds(0, 16)] = z
                    return c

                lax.fori_loop(0, CH, zea, 0)
                for k in range(NPT // CH):
                    pltpu.sync_copy(
                        eabuf, acc_e.at[pl.ds(sid * NPT + k * CH, CH)])

            @pl.when(cid == 1)
            def _():
                pltpu.sync_copy(onesz_hbm.at[1], ones)
                for k in range(NPT // CH):
                    pltpu.sync_copy(
                        ones.at[pl.ds(0, CH)],
                        acc_c.at[pl.ds(sid * NPT + k * CH, CH)])
                pltpu.sync_copy(onesz_hbm.at[0], ones)

        plsc.subcore_barrier()

        def chunk(j, c):
            pltpu.async_copy(x2_hbm.at[idx_s.at[j]], rows, sem).wait()
            pltpu.sync_copy(rows, acc_h.at[idx_d.at[j]], add=True)
            if with_meta:
                @pl.when(cid == 0)
                def _():
                    pltpu.sync_copy(ea_hbm.at[sid, j], eabuf)
                    pltpu.sync_copy(eabuf, acc_e.at[idx_d.at[j]], add=True)

                @pl.when(cid == 1)
                def _():
                    pltpu.sync_copy(ones.at[pl.ds(0, CH)],
                                    acc_c.at[idx_d.at[j]], add=True)
            return c

        lax.fori_loop(0, NCH, chunk, 0)
        plsc.subcore_barrier()

        sl = pl.ds(sid * NPT, NPT)
        pltpu.sync_copy(acc_h.at[sl], out_h.at[cid, sl])
        if with_meta:
            @pl.when(cid == 0)
            def _():
                pltpu.sync_copy(acc_e.at[sl], out_e.at[sl])

            @pl.when(cid == 1)
            def _():
                pltpu.sync_copy(acc_c.at[sl], out_c.at[sl])

    mesh = plsc.VectorSubcoreMesh(core_axis_name="c", subcore_axis_name="s")
    return pl.kernel(body, mesh=mesh, out_type=out_type, scratch_types=scratch,
                     compiler_params=pltpu.CompilerParams(
                         use_tc_tiling_on_sc=False))


_sc_pass_meta = _make_sc_pass(True)
_sc_pass = _make_sc_pass(False)

_RB = 1000


def _make_tc_fuse(split_out: bool):
    def fuse_body(hs, ph, pe, pc, wst, wsb, bs, wxt, wxb, we, bn, o):
        h_lo, h_hi = hs[0], hs[1]
        sh_lo = ph[0] + h_lo
        sh_hi = ph[1] + h_hi
        cnt = pc[...][:, :1] + 1.0
        dot = lambda a, b: jnp.dot(a, b, preferred_element_type=jnp.float32)
        num = dot(sh_lo, wxt[...]) + dot(sh_hi, wxb[...]) + dot(pe[...], we[...])
        self_ = dot(h_lo, wst[...]) + dot(h_hi, wsb[...])
        res = jnp.maximum(self_ + bs[...] + num / cnt + bn[...], 0.0)
        if split_out:
            o[0] = res[:, :HF]
            o[1] = res[:, HF:]
        else:
            o[...] = res

    grid = (N_NODES // _RB,)
    row = lambda i: (i, 0)
    srow = lambda i: (0, i, 0)
    fixed = lambda i: (0, 0)
    if split_out:
        out_spec = pl.BlockSpec((NC, _RB, HF), srow)
        out_shape = jax.ShapeDtypeStruct((NC, N_NODES, HF), jnp.float32)
    else:
        out_spec = pl.BlockSpec((_RB, D), row)
        out_shape = jax.ShapeDtypeStruct((N_NODES, D), jnp.float32)

    def call(hs, ph, pe, pc, W_self, b_self, W_neigh, b_neigh):
        return pl.pallas_call(
            fuse_body,
            grid=grid,
            in_specs=[
                pl.BlockSpec((NC, _RB, HF), srow),
                pl.BlockSpec((NC, _RB, HF), srow),
                pl.BlockSpec((_RB, DE), row),
                pl.BlockSpec((_RB, CW), row),
                pl.BlockSpec((HF, D), fixed),
                pl.BlockSpec((HF, D), fixed),
                pl.BlockSpec((1, D), fixed),
                pl.BlockSpec((HF, D), fixed),
                pl.BlockSpec((HF, D), fixed),
                pl.BlockSpec((DE, D), fixed),
                pl.BlockSpec((1, D), fixed),
            ],
            out_specs=out_spec,
            out_shape=out_shape,
        )(hs, ph, pe, pc, W_self[:HF], W_self[HF:], b_self.reshape(1, D),
          W_neigh[:HF], W_neigh[HF:D], W_neigh[D:], b_neigh.reshape(1, D))

    return call


_tc_fuse_split = _make_tc_fuse(True)
_tc_fuse_final = _make_tc_fuse(False)


def kernel(x, edge_index, edge_attr,
           W_self1, b_self1, W_neigh1, b_neigh1,
           W_self2, b_self2, W_neigh2, b_neigh2):
    src = edge_index[0].astype(jnp.int32).reshape(NS, NCH, CH)
    dst = edge_index[1].astype(jnp.int32).reshape(NS, NCH, CH)
    src2 = jnp.stack([src, src + N_NODES])
    ea = edge_attr.reshape(NS, NCH, CH, DE)
    onesz = jnp.stack([jnp.ones((128, CW), jnp.float32),
                       jnp.zeros((128, CW), jnp.float32)])

    xs = jnp.stack([x[:, :HF], x[:, HF:]])
    ph, pe, pc = _sc_pass_meta(xs.reshape(NC * N_NODES, HF), src2, dst,
                               ea, onesz)
    h1s = _tc_fuse_split(xs, ph, pe, pc, W_self1, b_self1, W_neigh1, b_neigh1)
    outs = _sc_pass(h1s.reshape(NC * N_NODES, HF), src2, dst)
    ph2 = outs[0] if isinstance(outs, (list, tuple)) else outs
    return _tc_fuse_final(h1s, ph2, pe, pc, W_self2, b_self2,
                          W_neigh2, b_neigh2)


# speedup vs baseline: 10.6631x; 1.5578x over previous
import jax
import jax.numpy as jnp
from jax import lax
from jax.experimental import pallas as pl
from jax.experimental.pallas import tpu as pltpu
from jax.experimental.pallas import tpu_sc as plsc

N_NODES = 10000
N_EDGES = 320000
D = 128
DE = 16
HF = 64

NC = 2
NS = 16
EPT = N_EDGES // NS
CH = 125
NCH = EPT // CH
NPT = N_NODES // NS
CW = 8
NBUF = 2


def _make_sc_pass(with_meta: bool):
    out_type = [jax.ShapeDtypeStruct((NC, N_NODES, HF), jnp.float32)]
    if with_meta:
        out_type += [
            jax.ShapeDtypeStruct((NC, N_NODES, DE), jnp.float32),
            jax.ShapeDtypeStruct((NC, N_NODES, CW), jnp.float32),
        ]
    scratch = [
        pltpu.VMEM((NCH, CH), jnp.int32),
        pltpu.VMEM((NCH, CH), jnp.int32),
        pltpu.VMEM((NBUF, CH, HF), jnp.float32),
        pltpu.VMEM_SHARED((N_NODES, HF), jnp.float32),
        pltpu.SemaphoreType.DMA,
        pltpu.SemaphoreType.DMA,
    ]
    if with_meta:
        scratch += [
            pltpu.VMEM((CH, DE), jnp.float32),
            pltpu.VMEM((128, CW), jnp.float32),
            pltpu.VMEM_SHARED((N_NODES, DE), jnp.float32),
            pltpu.VMEM_SHARED((N_NODES, CW), jnp.float32),
        ]

    def body(x2_hbm, src_hbm, dst_hbm, *rest):
        if with_meta:
            (ea_hbm, onesz_hbm, out_h, out_e, out_c,
             idx_s, idx_d, rows, acc_h, sem0, sem1,
             eabuf, ones, acc_e, acc_c) = rest
        else:
            (out_h, idx_s, idx_d, rows, acc_h, sem0, sem1) = rest
        sems = (sem0, sem1)

        cid = lax.axis_index("c")
        sid = lax.axis_index("s")

        pltpu.sync_copy(src_hbm.at[cid, sid], idx_s)
        pltpu.sync_copy(dst_hbm.at[sid], idx_d)

        z = jnp.zeros((16,), jnp.float32)

        def zrow(i, c):
            for k in range(HF // 16):
                rows[0, i, pl.ds(k * 16, 16)] = z
            return c

        lax.fori_loop(0, CH, zrow, 0)
        for k in range(NPT // CH):
            pltpu.sync_copy(rows.at[0],
                            acc_h.at[pl.ds(sid * NPT + k * CH, CH)])

        if with_meta:
            def zea(i, c):
                eabuf[i, pl.ds(0, 16)] = z
                return c

            lax.fori_loop(0, CH, zea, 0)
            for k in range(NPT // CH):
                pltpu.sync_copy(eabuf, acc_e.at[pl.ds(sid * NPT + k * CH, CH)])
            pltpu.sync_copy(onesz_hbm.at[1], ones)
            for k in range(NPT // CH):
                pltpu.sync_copy(ones.at[pl.ds(0, CH)],
                                acc_c.at[pl.ds(sid * NPT + k * CH, CH)])
            pltpu.sync_copy(onesz_hbm.at[0], ones)

        plsc.subcore_barrier()

        for b in range(NBUF):
            pltpu.async_copy(x2_hbm.at[idx_s.at[b]], rows.at[b], sems[b])

        def grp(g, c):
            for b in range(NBUF):
                j = g * NBUF + b
                pltpu.make_async_copy(
                    x2_hbm.at[idx_s.at[0]], rows.at[b], sems[b]).wait()
                pltpu.sync_copy(rows.at[b], acc_h.at[idx_d.at[j]], add=True)
                nxt = j + NBUF

                @pl.when(nxt < NCH)
                def _():
                    pltpu.async_copy(
                        x2_hbm.at[idx_s.at[nxt]], rows.at[b], sems[b])

                if with_meta:
                    in_first = j < (NCH // 2)
                    mine_ea = in_first == (cid == 0)

                    @pl.when(mine_ea)
                    def _():
                        pltpu.sync_copy(ea_hbm.at[sid, j], eabuf)
                        pltpu.sync_copy(eabuf, acc_e.at[idx_d.at[j]],
                                        add=True)

                    @pl.when(jnp.logical_not(mine_ea))
                    def _():
                        pltpu.sync_copy(ones.at[pl.ds(0, CH)],
                                        acc_c.at[idx_d.at[j]], add=True)
            return c

        lax.fori_loop(0, NCH // NBUF, grp, 0)
        plsc.subcore_barrier()

        sl = pl.ds(sid * NPT, NPT)
        pltpu.sync_copy(acc_h.at[sl], out_h.at[cid, sl])
        if with_meta:
            pltpu.sync_copy(acc_e.at[sl], out_e.at[cid, sl])
            pltpu.sync_copy(acc_c.at[sl], out_c.at[cid, sl])

    mesh = plsc.VectorSubcoreMesh(core_axis_name="c", subcore_axis_name="s")
    return pl.kernel(body, mesh=mesh, out_type=out_type, scratch_types=scratch,
                     compiler_params=pltpu.CompilerParams(
                         use_tc_tiling_on_sc=False))


_sc_pass_meta = _make_sc_pass(True)
_sc_pass = _make_sc_pass(False)

_RB = 1000


def _fuse_body(h, ph, pe0, pe1, pc0, pc1, wst, wsb, bs, wxt, wxb, we, bn, o):
    hv = h[...]
    h_lo, h_hi = hv[:, :HF], hv[:, HF:]
    sh_lo = ph[0] + h_lo
    sh_hi = ph[1] + h_hi
    cnt = pc0[0][:, :1] + pc1[0][:, :1] + 1.0
    dot = lambda a, b: jnp.dot(a, b, preferred_element_type=jnp.float32)
    num = (dot(sh_lo, wxt[...]) + dot(sh_hi, wxb[...])
           + dot(pe0[0] + pe1[0], we[...]))
    self_ = dot(h_lo, wst[...]) + dot(h_hi, wsb[...])
    o[...] = jnp.maximum(self_ + bs[...] + num / cnt + bn[...], 0.0)


def _tc_fuse(h, ph, pe, pc, W_self, b_self, W_neigh, b_neigh):
    grid = (N_NODES // _RB,)
    row = lambda i: (i, 0)
    part0 = lambda i: (0, i, 0)
    part1 = lambda i: (1, i, 0)
    fixed = lambda i: (0, 0)
    return pl.pallas_call(
        _fuse_body,
        grid=grid,
        in_specs=[
            pl.BlockSpec((_RB, D), row),
            pl.BlockSpec((NC, _RB, HF), part0),
            pl.BlockSpec((1, _RB, DE), part0),
            pl.BlockSpec((1, _RB, DE), part1),
            pl.BlockSpec((1, _RB, CW), part0),
            pl.BlockSpec((1, _RB, CW), part1),
            pl.BlockSpec((HF, D), fixed),
            pl.BlockSpec((HF, D), fixed),
            pl.BlockSpec((1, D), fixed),
            pl.BlockSpec((HF, D), fixed),
            pl.BlockSpec((HF, D), fixed),
            pl.BlockSpec((DE, D), fixed),
            pl.BlockSpec((1, D), fixed),
        ],
        out_specs=pl.BlockSpec((_RB, D), row),
        out_shape=jax.ShapeDtypeStruct((N_NODES, D), jnp.float32),
    )(h, ph, pe, pe, pc, pc, W_self[:HF], W_self[HF:], b_self.reshape(1, D),
      W_neigh[:HF], W_neigh[HF:D], W_neigh[D:], b_neigh.reshape(1, D))


def kernel(x, edge_index, edge_attr,
           W_self1, b_self1, W_neigh1, b_neigh1,
           W_self2, b_self2, W_neigh2, b_neigh2):
    src = edge_index[0].astype(jnp.int32)
    dst = edge_index[1].astype(jnp.int32).reshape(NS, NCH, CH)
    src2 = jnp.stack([2 * src, 2 * src + 1]).reshape(NC, NS, NCH, CH)
    ea = edge_attr.reshape(NS, NCH, CH, DE)
    onesz = jnp.stack([jnp.ones((128, CW), jnp.float32),
                       jnp.zeros((128, CW), jnp.float32)])

    ph, pe, pc = _sc_pass_meta(x.reshape(NC * N_NODES, HF), src2, dst,
                               ea, onesz)
    h1 = _tc_fuse(x, ph, pe, pc, W_self1, b_self1, W_neigh1, b_neigh1)
    outs = _sc_pass(h1.reshape(NC * N_NODES, HF), src2, dst)
    ph2 = outs[0] if isinstance(outs, (list, tuple)) else outs
    return _tc_fuse(h1, ph2, pe, pc, W_self2, b_self2, W_neigh2, b_neigh2)
